# baseline (speedup 1.0000x reference)
import functools

import jax
import jax.numpy as jnp
from jax import lax
from jax.experimental import pallas as pl
from jax.experimental.pallas import tpu as pltpu
from jax.experimental.pallas import tpu_sc as plsc

_TI = 512
_L = 16


def _combine_body(idx_hbm, w_hbm, c_hbm, idx_v, w_v, c_v, sem):
    on_worker0 = jnp.logical_and(lax.axis_index("c") == 0,
                                 lax.axis_index("s") == 0)

    @pl.when(on_worker0)
    def _():
        pltpu.sync_copy(idx_hbm, idx_v)
        pltpu.sync_copy(w_hbm, w_v)
        for j in range(256 // _L):
            c_v[pl.ds(j * _L, _L)] = jnp.zeros((_L,), jnp.float32)
        lanes = lax.iota(jnp.int32, _L)
        for j in range(64 // _L):
            ev = idx_v[pl.ds(j * _L, _L)]
            wv = w_v[pl.ds(j * _L, _L)]
            tok = j * (_L // 2) + lax.shift_right_logical(lanes, 1)
            flat = ev * 32 + tok
            for k in range(2):
                mask = lax.bitwise_and(lanes, 1) == k
                plsc.addupdate_scatter(c_v, [flat], wv, mask=mask)
        pltpu.sync_copy(c_v, c_hbm)


def _combine_coeffs(idx_flat, w_flat):
    mesh = plsc.VectorSubcoreMesh(core_axis_name="c", subcore_axis_name="s")
    f = pl.kernel(
        _combine_body,
        mesh=mesh,
        out_type=jax.ShapeDtypeStruct((256,), jnp.float32),
        scratch_types=[
            pltpu.VMEM((64,), jnp.int32),
            pltpu.VMEM((64,), jnp.float32),
            pltpu.VMEM((256,), jnp.float32),
            pltpu.SemaphoreType.DMA,
        ],
        compiler_params=pltpu.CompilerParams(needs_layout_passes=False),
    )
    return f(idx_flat, w_flat).reshape(8, 32)


def _moe_body(c_ref, x_ref, g_ref, u_ref, d_ref, out_ref, xt_scr, acc_scr):
    e = pl.program_id(0)
    i = pl.program_id(1)
    first = jnp.logical_and(e == 0, i == 0)
    last = jnp.logical_and(e == pl.num_programs(0) - 1,
                           i == pl.num_programs(1) - 1)

    @pl.when(first)
    def _init():
        xt_scr[...] = x_ref[...].T

    xt = xt_scr[...]
    g = jax.lax.dot_general(g_ref[0], xt, (((1,), (0,)), ((), ())),
                            preferred_element_type=jnp.float32)
    u = jax.lax.dot_general(u_ref[0], xt, (((1,), (0,)), ((), ())),
                            preferred_element_type=jnp.float32)
    h = (g * jax.nn.sigmoid(g)) * u

    rows = lax.broadcasted_iota(jnp.int32, c_ref.shape, 0)
    ce = jnp.sum(jnp.where(rows == e, c_ref[...], 0.0), axis=0)
    h = h * ce[None, :]

    contrib = jax.lax.dot_general(d_ref[0], h, (((1,), (0,)), ((), ())),
                                  preferred_element_type=jnp.float32)

    @pl.when(first)
    def _set():
        acc_scr[...] = contrib

    @pl.when(jnp.logical_not(first))
    def _add():
        acc_scr[...] += contrib

    @pl.when(last)
    def _emit():
        out_ref[...] = acc_scr[...].T


@functools.partial(jax.jit, static_argnames=())
def kernel(x, expert_indices, expert_weights, gate_proj, up_proj, down_proj):
    batch, seq_len, hidden = x.shape
    num_experts = gate_proj.shape[0]
    inter = gate_proj.shape[1]
    top_k = expert_indices.shape[-1]
    num_tokens = batch * seq_len

    x2 = x.reshape(num_tokens, hidden)
    idx_flat = expert_indices.reshape(num_tokens * top_k)
    w_flat = expert_weights.reshape(num_tokens * top_k)

    c = _combine_coeffs(idx_flat, w_flat)

    n_i = inter // _TI
    grid = (num_experts, n_i)

    out = pl.pallas_call(
        _moe_body,
        grid=grid,
        in_specs=[
            pl.BlockSpec((num_experts, num_tokens), lambda e, i: (0, 0)),
            pl.BlockSpec((num_tokens, hidden), lambda e, i: (0, 0)),
            pl.BlockSpec((1, _TI, hidden), lambda e, i: (e, i, 0)),
            pl.BlockSpec((1, _TI, hidden), lambda e, i: (e, i, 0)),
            pl.BlockSpec((1, hidden, _TI), lambda e, i: (e, 0, i)),
        ],
        out_specs=pl.BlockSpec((num_tokens, hidden), lambda e, i: (0, 0)),
        out_shape=jax.ShapeDtypeStruct((num_tokens, hidden), jnp.float32),
        scratch_shapes=[
            pltpu.VMEM((hidden, num_tokens), jnp.float32),
            pltpu.VMEM((hidden, num_tokens), jnp.float32),
        ],
    )(c, x2, gate_proj, up_proj, down_proj)

    return out.reshape(batch, seq_len, hidden)
